# Initial kernel scaffold; baseline (speedup 1.0000x reference)
import functools

import jax
import jax.numpy as jnp
from jax import lax
from jax.experimental import pallas as pl
from jax.experimental.pallas import tpu as pltpu
from jax.experimental.pallas import tpu_sc as plsc

N = 10000
E = 320000
DIN = 128
DH = 32
DOUT = 20

NW = 32
K = 4
CBN = DH // K
G = NW // CBN
EG = E // G
C = 2000
EW = E // NW
L = 16


def _seg_body(with_counts, *refs):
    if with_counts:
        (y_hbm, src_hbm, dst_hbm, zer_hbm,
         part_hbm, cnt_hbm, table, acc, srcb, dstb, cntv) = refs
    else:
        (y_hbm, src_hbm, dst_hbm, zer_hbm,
         part_hbm, table, acc, srcb, dstb) = refs

    c = lax.axis_index("c")
    s = lax.axis_index("s")
    wid = s * 2 + c
    cb = wid % CBN
    g = wid // CBN
    c0 = cb * K

    pltpu.sync_copy(y_hbm.at[:, pl.ds(c0, K)], table)
    pltpu.sync_copy(zer_hbm, acc)

    ebase = g * EG

    def chunk_body(j, _):
        off = ebase + j * C
        pltpu.sync_copy(src_hbm.at[pl.ds(off, C)], srcb)
        pltpu.sync_copy(dst_hbm.at[pl.ds(off, C)], dstb)

        def inner(i, _):
            s16 = srcb[pl.ds(i * L, L)]
            d16 = dstb[pl.ds(i * L, L)]
            for k in range(K):
                kv = jnp.full((L,), k, jnp.int32)
                vals = plsc.load_gather(table, [s16, kv])
                plsc.addupdate_scatter(acc, [d16, kv], vals)
            return 0

        lax.fori_loop(0, C // L, inner, 0)
        return 0

    lax.fori_loop(0, EG // C, chunk_body, 0)
    pltpu.sync_copy(acc, part_hbm.at[g, :, pl.ds(c0, K)])

    if with_counts:
        def zbody(i, _):
            cntv[pl.ds(i * L, L)] = jnp.zeros((L,), jnp.float32)
            return 0

        lax.fori_loop(0, N // L, zbody, 0)
        ones = jnp.full((L,), 1.0, jnp.float32)
        cbase = wid * EW

        def cchunk(j, _):
            pltpu.sync_copy(dst_hbm.at[pl.ds(cbase + j * C, C)], dstb)

            def cinner(i, _):
                d16 = dstb[pl.ds(i * L, L)]
                plsc.addupdate_scatter(cntv, [d16], ones)
                return 0

            lax.fori_loop(0, C // L, cinner, 0)
            return 0

        lax.fori_loop(0, EW // C, cchunk, 0)
        pltpu.sync_copy(cntv, cnt_hbm.at[:, wid])


def _make_seg(with_counts):
    mesh = plsc.VectorSubcoreMesh(core_axis_name="c", subcore_axis_name="s")
    out_type = [jax.ShapeDtypeStruct((G, N, DH), jnp.float32)]
    if with_counts:
        out_type.append(jax.ShapeDtypeStruct((N, NW), jnp.float32))
    scratch = [
        pltpu.VMEM((N, K), jnp.float32),
        pltpu.VMEM((N, K), jnp.float32),
        pltpu.VMEM((C,), jnp.int32),
        pltpu.VMEM((C,), jnp.int32),
    ]
    if with_counts:
        scratch.append(pltpu.VMEM((N,), jnp.float32))
    return pl.kernel(
        functools.partial(_seg_body, with_counts),
        out_type=tuple(out_type) if with_counts else out_type[0],
        mesh=mesh,
        scratch_types=scratch,
    )


_seg_with_counts = _make_seg(True)
_seg_no_counts = _make_seg(False)


_R = 2000


def _stage1(x, w1t):
    def body(xr, wr, outr):
        outr[...] = jnp.dot(xr[...], wr[...], preferred_element_type=jnp.float32)

    return pl.pallas_call(
        body,
        grid=(N // _R,),
        in_specs=[pl.BlockSpec((_R, DIN), lambda i: (i, 0)),
                  pl.BlockSpec((DIN, 2 * DH), lambda i: (0, 0))],
        out_specs=pl.BlockSpec((_R, 2 * DH), lambda i: (i, 0)),
        out_shape=jax.ShapeDtypeStruct((N, 2 * DH), jnp.float32),
    )(x, w1t)


def _post_agg(pr, cr, yr, blr):
    psum = jnp.sum(pr[...], axis=0)
    cs = jnp.sum(cr[...], axis=1, keepdims=True)
    mean = psum / jnp.maximum(cs, 1.0)
    t = mean + blr[...] + yr[...][:, DH:]
    nrm = jnp.sqrt(jnp.sum(t * t, axis=1, keepdims=True))
    return jnp.tanh(t / jnp.maximum(nrm, 1e-12))


def _stage_mid(part, cnt, y, bl, w2t):
    def body(pr, cr, yr, blr, wr, outr):
        h = _post_agg(pr, cr, yr, blr)
        outr[...] = jnp.dot(h, wr[...], preferred_element_type=jnp.float32)

    return pl.pallas_call(
        body,
        grid=(N // _R,),
        in_specs=[pl.BlockSpec((G, _R, DH), lambda i: (0, i, 0)),
                  pl.BlockSpec((_R, NW), lambda i: (i, 0)),
                  pl.BlockSpec((_R, 2 * DH), lambda i: (i, 0)),
                  pl.BlockSpec((1, DH), lambda i: (0, 0)),
                  pl.BlockSpec((DH, 2 * DH), lambda i: (0, 0))],
        out_specs=pl.BlockSpec((_R, 2 * DH), lambda i: (i, 0)),
        out_shape=jax.ShapeDtypeStruct((N, 2 * DH), jnp.float32),
    )(part, cnt, y, bl, w2t)


def _stage_out(part, cnt, y, bl, wct, bc):
    def body(pr, cr, yr, blr, wr, bcr, outr, hr):
        h = _post_agg(pr, cr, yr, blr)
        hr[...] = h
        logits = jnp.dot(h, wr[...], preferred_element_type=jnp.float32) + bcr[...]
        m = jnp.max(logits, axis=1, keepdims=True)
        lse = jnp.log(jnp.sum(jnp.exp(logits - m), axis=1, keepdims=True)) + m
        outr[...] = logits - lse

    return pl.pallas_call(
        body,
        grid=(N // _R,),
        in_specs=[pl.BlockSpec((G, _R, DH), lambda i: (0, i, 0)),
                  pl.BlockSpec((_R, NW), lambda i: (i, 0)),
                  pl.BlockSpec((_R, 2 * DH), lambda i: (i, 0)),
                  pl.BlockSpec((1, DH), lambda i: (0, 0)),
                  pl.BlockSpec((DH, DOUT), lambda i: (0, 0)),
                  pl.BlockSpec((1, DOUT), lambda i: (0, 0))],
        out_specs=[pl.BlockSpec((_R, DOUT), lambda i: (i, 0)),
                   pl.BlockSpec((_R, DH), lambda i: (i, 0))],
        out_shape=[jax.ShapeDtypeStruct((N, DOUT), jnp.float32),
                   jax.ShapeDtypeStruct((N, DH), jnp.float32)],
    )(part, cnt, y, bl, wct, bc)


def kernel(x, edge_index, Wl1, bl1, Wr1, Wl2, bl2, Wr2, Wc, bc):
    ei = edge_index.astype(jnp.int32)
    src = ei[0]
    dst = ei[1]
    w1t = jnp.concatenate([Wl1, Wr1], axis=0).T
    w2t = jnp.concatenate([Wl2, Wr2], axis=0).T
    wct = Wc.T
    zer = jnp.zeros((N, K), jnp.float32)

    y1 = _stage1(x, w1t)
    part1, cntp = _seg_with_counts(y1, src, dst, zer)
    y2 = _stage_mid(part1, cntp, y1, bl1.reshape(1, -1), w2t)
    part2 = _seg_no_counts(y2, src, dst, zer)
    logp, h = _stage_out(part2, cntp, y2, bl2.reshape(1, -1), wct, bc.reshape(1, -1))
    return (logp, h)


# speedup vs baseline: 8.1200x; 8.1200x over previous
import functools

import jax
import jax.numpy as jnp
from jax import lax
from jax.experimental import pallas as pl
from jax.experimental.pallas import tpu as pltpu
from jax.experimental.pallas import tpu_sc as plsc

N = 10000
E = 320000
DIN = 128
DH = 32
DOUT = 20

NW = 32
K = 4
CBN = DH // K
G = NW // CBN
EG = E // G
C = 2000
EW = E // NW
L = 16


def _seg_body(with_counts, *refs):
    if with_counts:
        (y_hbm, src_hbm, dst_hbm, zer_hbm,
         part_hbm, cnt_hbm, table, acc, srcb, dstb, cntv) = refs
    else:
        (y_hbm, src_hbm, dst_hbm, zer_hbm,
         part_hbm, table, acc, srcb, dstb) = refs

    c = lax.axis_index("c")
    s = lax.axis_index("s")
    wid = s * 2 + c
    cb = wid % CBN
    g = wid // CBN
    c0 = cb * K

    pltpu.sync_copy(y_hbm.at[pl.ds(c0, K)], table)
    pltpu.sync_copy(zer_hbm, acc)

    ebase = g * EG

    def chunk_body(j, _):
        off = ebase + j * C
        pltpu.sync_copy(src_hbm.at[pl.ds(off, C)], srcb)
        pltpu.sync_copy(dst_hbm.at[pl.ds(off, C)], dstb)

        def inner(i, _):
            s16 = srcb[pl.ds(i * L, L)]
            d16 = dstb[pl.ds(i * L, L)]
            for k in range(K):
                kv = jnp.full((L,), k, jnp.int32)
                vals = plsc.load_gather(table, [kv, s16])
                plsc.addupdate_scatter(acc, [kv, d16], vals)
            return 0

        lax.fori_loop(0, C // L, inner, 0)
        return 0

    lax.fori_loop(0, EG // C, chunk_body, 0)
    pltpu.sync_copy(acc, part_hbm.at[g, pl.ds(c0, K)])

    if with_counts:
        def zbody(i, _):
            cntv[pl.ds(i * L, L)] = jnp.zeros((L,), jnp.float32)
            return 0

        lax.fori_loop(0, N // L, zbody, 0)
        ones = jnp.full((L,), 1.0, jnp.float32)
        cbase = wid * EW

        def cchunk(j, _):
            pltpu.sync_copy(dst_hbm.at[pl.ds(cbase + j * C, C)], dstb)

            def cinner(i, _):
                d16 = dstb[pl.ds(i * L, L)]
                plsc.addupdate_scatter(cntv, [d16], ones)
                return 0

            lax.fori_loop(0, C // L, cinner, 0)
            return 0

        lax.fori_loop(0, EW // C, cchunk, 0)
        pltpu.sync_copy(cntv, cnt_hbm.at[wid])


def _make_seg(with_counts):
    mesh = plsc.VectorSubcoreMesh(core_axis_name="c", subcore_axis_name="s")
    out_type = [jax.ShapeDtypeStruct((G, DH, N), jnp.float32)]
    if with_counts:
        out_type.append(jax.ShapeDtypeStruct((NW, N), jnp.float32))
    scratch = [
        pltpu.VMEM((K, N), jnp.float32),
        pltpu.VMEM((K, N), jnp.float32),
        pltpu.VMEM((C,), jnp.int32),
        pltpu.VMEM((C,), jnp.int32),
    ]
    if with_counts:
        scratch.append(pltpu.VMEM((N,), jnp.float32))
    return pl.kernel(
        functools.partial(_seg_body, with_counts),
        out_type=tuple(out_type) if with_counts else out_type[0],
        mesh=mesh,
        scratch_types=scratch,
        compiler_params=pltpu.CompilerParams(use_tc_tiling_on_sc=False,
                                             needs_layout_passes=False),
    )


_seg_with_counts = _make_seg(True)
_seg_no_counts = _make_seg(False)


_R = 2048
_NSTEP = (N + _R - 1) // _R


def _stage1(x, w1c):
    def body(xr, wr, outr):
        outr[...] = lax.dot_general(wr[...], xr[...],
                                    (((1,), (1,)), ((), ())),
                                    preferred_element_type=jnp.float32)

    return pl.pallas_call(
        body,
        grid=(_NSTEP,),
        in_specs=[pl.BlockSpec((_R, DIN), lambda i: (i, 0)),
                  pl.BlockSpec((2 * DH, DIN), lambda i: (0, 0))],
        out_specs=pl.BlockSpec((2 * DH, _R), lambda i: (0, i)),
        out_shape=jax.ShapeDtypeStruct((2 * DH, N), jnp.float32),
    )(x, w1c)


def _post_agg(pr, cr, yr, blr):
    psum = jnp.sum(pr[...], axis=0)
    cs = jnp.dot(jnp.ones((1, NW), jnp.float32), cr[...],
                 preferred_element_type=jnp.float32)
    mean = psum / jnp.maximum(cs, 1.0)
    t = mean + blr[...] + yr[...][DH:, :]
    nrm = jnp.sqrt(jnp.sum(t * t, axis=0, keepdims=True))
    return jnp.tanh(t / jnp.maximum(nrm, 1e-12))


def _stage_mid(part, cnt, y, bl, w2c):
    def body(pr, cr, yr, blr, wr, outr):
        h = _post_agg(pr, cr, yr, blr)
        outr[...] = jnp.dot(wr[...], h, preferred_element_type=jnp.float32)

    return pl.pallas_call(
        body,
        grid=(_NSTEP,),
        in_specs=[pl.BlockSpec((G, DH, _R), lambda i: (0, 0, i)),
                  pl.BlockSpec((NW, _R), lambda i: (0, i)),
                  pl.BlockSpec((2 * DH, _R), lambda i: (0, i)),
                  pl.BlockSpec((DH, 1), lambda i: (0, 0)),
                  pl.BlockSpec((2 * DH, DH), lambda i: (0, 0))],
        out_specs=pl.BlockSpec((2 * DH, _R), lambda i: (0, i)),
        out_shape=jax.ShapeDtypeStruct((2 * DH, N), jnp.float32),
    )(part, cnt, y, bl, w2c)


def _stage_out(part, cnt, y, bl, wc, bc):
    def body(pr, cr, yr, blr, wr, bcr, outr, hr):
        h = _post_agg(pr, cr, yr, blr)
        hr[...] = h
        logits = jnp.dot(wr[...], h, preferred_element_type=jnp.float32) + bcr[...]
        m = jnp.max(logits, axis=0, keepdims=True)
        lse = jnp.log(jnp.sum(jnp.exp(logits - m), axis=0, keepdims=True)) + m
        outr[...] = logits - lse

    return pl.pallas_call(
        body,
        grid=(_NSTEP,),
        in_specs=[pl.BlockSpec((G, DH, _R), lambda i: (0, 0, i)),
                  pl.BlockSpec((NW, _R), lambda i: (0, i)),
                  pl.BlockSpec((2 * DH, _R), lambda i: (0, i)),
                  pl.BlockSpec((DH, 1), lambda i: (0, 0)),
                  pl.BlockSpec((DOUT, DH), lambda i: (0, 0)),
                  pl.BlockSpec((DOUT, 1), lambda i: (0, 0))],
        out_specs=[pl.BlockSpec((DOUT, _R), lambda i: (0, i)),
                   pl.BlockSpec((DH, _R), lambda i: (0, i))],
        out_shape=[jax.ShapeDtypeStruct((DOUT, N), jnp.float32),
                   jax.ShapeDtypeStruct((DH, N), jnp.float32)],
    )(part, cnt, y, bl, wc, bc)


def kernel(x, edge_index, Wl1, bl1, Wr1, Wl2, bl2, Wr2, Wc, bc):
    ei = edge_index.astype(jnp.int32)
    src = ei[0]
    dst = ei[1]
    w1c = jnp.concatenate([Wl1, Wr1], axis=0)
    w2c = jnp.concatenate([Wl2, Wr2], axis=0)
    zer = jnp.zeros((K, N), jnp.float32)

    y1t = _stage1(x, w1c)
    part1, cntp = _seg_with_counts(y1t, src, dst, zer)
    y2t = _stage_mid(part1, cntp, y1t, bl1.reshape(-1, 1), w2c)
    part2 = _seg_no_counts(y2t, src, dst, zer)
    logp_t, h_t = _stage_out(part2, cntp, y2t, bl2.reshape(-1, 1),
                             Wc, bc.reshape(-1, 1))
    return (logp_t.T, h_t.T)
